# baseline (speedup 1.0000x reference)
import functools

import jax
import jax.numpy as jnp
from jax import lax
from jax.experimental import pallas as pl
from jax.experimental.pallas import tpu as pltpu
from jax.experimental.pallas import tpu_sc as plsc

N = 10000
D = 128
EPS = 1e-5

NC = 2
NS = 16
NW = NC * NS
LANES = 16
BATCH = 128
NSLOT = 4
NSLOT_A = 2

N_ACC = 10240
NDUM = N_ACC - N
RPT = N_ACC // NS

BLK = 1000

_mesh = plsc.VectorSubcoreMesh(
    core_axis_name="c", subcore_axis_name="s", num_cores=NC, num_subcores=NS)


def _mask_cols(row_v, col_v, colm_v):
  for j in range(BATCH // LANES):
    sl = pl.ds(j * LANES, LANES)
    r = row_v[sl]
    c = col_v[sl]
    dummy = jnp.int32(N) + lax.rem(r, jnp.int32(NDUM))
    colm_v[sl] = jnp.where(r == c, dummy, c)


def _deg_body(niter, epw, ei, zeros128, ones128, out, *scr):
  row_v = scr[0:NSLOT]
  col_v = scr[NSLOT:2 * NSLOT]
  colm_v = scr[2 * NSLOT:3 * NSLOT]
  ones_v = scr[3 * NSLOT]
  acc = scr[3 * NSLOT + 1]
  sem_i = scr[3 * NSLOT + 2:3 * NSLOT + 2 + NSLOT]
  sem_s = scr[3 * NSLOT + 2 + NSLOT:3 * NSLOT + 2 + 2 * NSLOT]

  cid = lax.axis_index("c")
  sid = lax.axis_index("s")
  wid = cid * NS + sid
  my_rows = pl.ds(sid * RPT, RPT)
  pltpu.sync_copy(zeros128, acc.at[my_rows])
  pltpu.sync_copy(ones128, ones_v)
  plsc.subcore_barrier()

  def ebase(b):
    return pl.multiple_of(wid * epw + b * BATCH, BATCH)

  def fire_idx(b, j):
    base = ebase(b)
    pltpu.async_copy(ei.at[0, pl.ds(base, BATCH)], row_v[j], sem_i[j])
    pltpu.async_copy(ei.at[1, pl.ds(base, BATCH)], col_v[j], sem_i[j])

  def wait_idx(b, j):
    base = ebase(b)
    pltpu.make_async_copy(ei.at[0, pl.ds(base, BATCH)], row_v[j], sem_i[j]).wait()
    pltpu.make_async_copy(ei.at[1, pl.ds(base, BATCH)], col_v[j], sem_i[j]).wait()

  for j in range(NSLOT):
    fire_idx(j, j)

  def body(i, carry):
    b0 = i * NSLOT
    scats = []
    for j in range(NSLOT):
      wait_idx(b0 + j, j)
      _mask_cols(row_v[j], col_v[j], colm_v[j])
      scats.append(
          pltpu.async_copy(ones_v, acc.at[colm_v[j]], sem_s[j], add=True))

      @pl.when(i < niter - 1)
      def _(j=j):
        fire_idx(b0 + NSLOT + j, j)

    for dsc in scats:
      dsc.wait()
    return carry

  lax.fori_loop(0, niter, body, 0)
  plsc.subcore_barrier()
  pltpu.sync_copy(acc.at[my_rows], out.at[cid, my_rows])


def _agg_body(niter, epw, hp, ei, zeros128, out, *scr):
  ns = NSLOT_A
  row_v = scr[0:ns]
  col_v = scr[ns:2 * ns]
  colm_v = scr[2 * ns:3 * ns]
  rows_v = scr[3 * ns:4 * ns]
  acc = scr[4 * ns]
  sem_i = scr[4 * ns + 1:4 * ns + 1 + ns]
  sem_g = scr[4 * ns + 1 + ns:4 * ns + 1 + 2 * ns]
  sem_s = scr[4 * ns + 1 + 2 * ns:4 * ns + 1 + 3 * ns]

  cid = lax.axis_index("c")
  sid = lax.axis_index("s")
  wid = cid * NS + sid
  my_rows = pl.ds(sid * RPT, RPT)
  pltpu.sync_copy(zeros128, acc.at[my_rows])
  plsc.subcore_barrier()

  def ebase(b):
    return pl.multiple_of(wid * epw + b * BATCH, BATCH)

  def fire_idx(b, j):
    base = ebase(b)
    pltpu.async_copy(ei.at[0, pl.ds(base, BATCH)], row_v[j], sem_i[j])
    pltpu.async_copy(ei.at[1, pl.ds(base, BATCH)], col_v[j], sem_i[j])

  def wait_idx(b, j):
    base = ebase(b)
    pltpu.make_async_copy(ei.at[0, pl.ds(base, BATCH)], row_v[j], sem_i[j]).wait()
    pltpu.make_async_copy(ei.at[1, pl.ds(base, BATCH)], col_v[j], sem_i[j]).wait()

  for j in range(ns):
    fire_idx(j, j)

  def body(i, carry):
    b0 = i * ns
    gats = []
    for j in range(ns):
      wait_idx(b0 + j, j)
      _mask_cols(row_v[j], col_v[j], colm_v[j])
      gats.append(pltpu.async_copy(hp.at[row_v[j]], rows_v[j], sem_g[j]))

    scats = []
    for j in range(ns):
      gats[j].wait()
      scats.append(
          pltpu.async_copy(rows_v[j], acc.at[colm_v[j]], sem_s[j], add=True))

      @pl.when(i < niter - 1)
      def _(j=j):
        fire_idx(b0 + ns + j, j)

    for dsc in scats:
      dsc.wait()
    return carry

  lax.fori_loop(0, niter, body, 0)
  plsc.subcore_barrier()
  pltpu.sync_copy(acc.at[my_rows], out.at[cid, my_rows])


def _dis_body(degp_ref, out_ref):
  p = degp_ref[...]
  deg = jnp.sum(p[0], axis=-1) + jnp.sum(p[1], axis=-1) + 1.0
  out_ref[...] = lax.rsqrt(deg)[:, None]


def _mm_scale_body(x_ref, w_ref, dis_ref, out_ref):
  h = jnp.dot(x_ref[...], w_ref[...], preferred_element_type=jnp.float32)
  out_ref[...] = h * dis_ref[...]


def _norm_body(aggp_ref, hp_ref, dis_ref, b_ref, g_ref, be_ref, out_ref):
  a = aggp_ref[0] + aggp_ref[1] + hp_ref[...]
  t = a * dis_ref[...] + b_ref[...]
  r = jnp.maximum(t, 0.0)
  mu = jnp.mean(r, axis=-1, keepdims=True)
  c = r - mu
  var = jnp.mean(c * c, axis=-1, keepdims=True)
  out_ref[...] = c * lax.rsqrt(var + EPS) * g_ref[...] + be_ref[...]


def _make_sc_kernels(e_pad):
  epw = e_pad // NW
  deg_k = functools.partial(
      pl.kernel,
      out_type=jax.ShapeDtypeStruct((NC, N_ACC, D), jnp.float32),
      mesh=_mesh,
      scratch_types=(
          [pltpu.VMEM((BATCH,), jnp.int32)] * (3 * NSLOT)
          + [pltpu.VMEM((BATCH, D), jnp.float32),
             pltpu.VMEM_SHARED((N_ACC, D), jnp.float32)]
          + [pltpu.SemaphoreType.DMA] * (2 * NSLOT)
      ))(functools.partial(_deg_body, epw // (BATCH * NSLOT), epw))
  agg_k = functools.partial(
      pl.kernel,
      out_type=jax.ShapeDtypeStruct((NC, N_ACC, D), jnp.float32),
      mesh=_mesh,
      scratch_types=(
          [pltpu.VMEM((BATCH,), jnp.int32)] * (3 * NSLOT_A)
          + [pltpu.VMEM((BATCH, D), jnp.float32)] * NSLOT_A
          + [pltpu.VMEM_SHARED((N_ACC, D), jnp.float32)]
          + [pltpu.SemaphoreType.DMA] * (3 * NSLOT_A)
      ))(functools.partial(_agg_body, epw // (BATCH * NSLOT_A), epw))
  return deg_k, agg_k


def _tc_dis(degp):
  return pl.pallas_call(
      _dis_body,
      grid=(N // BLK,),
      in_specs=[pl.BlockSpec((NC, BLK, D), lambda i: (0, i, 0))],
      out_specs=pl.BlockSpec((BLK, 1), lambda i: (i, 0)),
      out_shape=jax.ShapeDtypeStruct((N, 1), jnp.float32),
  )(degp)


def _tc_matmul_scale(x, w, dis):
  return pl.pallas_call(
      _mm_scale_body,
      grid=(N // BLK,),
      in_specs=[
          pl.BlockSpec((BLK, D), lambda i: (i, 0)),
          pl.BlockSpec((D, D), lambda i: (0, 0)),
          pl.BlockSpec((BLK, 1), lambda i: (i, 0)),
      ],
      out_specs=pl.BlockSpec((BLK, D), lambda i: (i, 0)),
      out_shape=jax.ShapeDtypeStruct((N, D), jnp.float32),
  )(x, w, dis)


def _tc_norm(aggp, hp, dis, b, g, be):
  return pl.pallas_call(
      _norm_body,
      grid=(N // BLK,),
      in_specs=[
          pl.BlockSpec((NC, BLK, D), lambda i: (0, i, 0)),
          pl.BlockSpec((BLK, D), lambda i: (i, 0)),
          pl.BlockSpec((BLK, 1), lambda i: (i, 0)),
          pl.BlockSpec((1, D), lambda i: (0, 0)),
          pl.BlockSpec((1, D), lambda i: (0, 0)),
          pl.BlockSpec((1, D), lambda i: (0, 0)),
      ],
      out_specs=pl.BlockSpec((BLK, D), lambda i: (i, 0)),
      out_shape=jax.ShapeDtypeStruct((N, D), jnp.float32),
  )(aggp, hp, dis, b, g, be)


def kernel(x, edge_index, W1, b1, g1, be1, W2, b2, g2, be2):
  e = edge_index.shape[1]
  chunk = NW * BATCH * NSLOT
  e_pad = ((e + chunk - 1) // chunk) * chunk
  pad = e_pad - e
  ei = edge_index.astype(jnp.int32)
  if pad:
    fill = jnp.stack([
        jnp.zeros((pad,), jnp.int32),
        N + (jnp.arange(pad, dtype=jnp.int32) % NDUM)])
    ei = jnp.concatenate([ei, fill], axis=1)

  ones128 = jnp.full((BATCH, D), 1.0 / D, jnp.float32)
  zeros128 = jnp.zeros((RPT, D), jnp.float32)
  b1r, g1r, be1r = b1.reshape(1, D), g1.reshape(1, D), be1.reshape(1, D)
  b2r, g2r, be2r = b2.reshape(1, D), g2.reshape(1, D), be2.reshape(1, D)

  deg_k, agg_k = _make_sc_kernels(e_pad)

  degp = deg_k(ei, zeros128, ones128)
  dis = _tc_dis(degp)

  h1 = _tc_matmul_scale(x, W1, dis)
  agg1 = agg_k(h1, ei, zeros128)
  z1 = _tc_norm(agg1, h1, dis, b1r, g1r, be1r)

  h2 = _tc_matmul_scale(z1, W2, dis)
  agg2 = agg_k(h2, ei, zeros128)
  z2 = _tc_norm(agg2, h2, dis, b2r, g2r, be2r)
  return z2


# speedup vs baseline: 1.2737x; 1.2737x over previous
import functools

import jax
import jax.numpy as jnp
from jax import lax
from jax.experimental import pallas as pl
from jax.experimental.pallas import tpu as pltpu
from jax.experimental.pallas import tpu_sc as plsc

N = 10000
D = 128
EPS = 1e-5

NC = 2
NS = 16
NW = NC * NS
LANES = 16
BATCH = 128
NSLOT = 4
NSLOT_A = 2

N_ACC = 10240
NDUM = N_ACC - N
RPT = N_ACC // NS

BLK = 1000

_mesh = plsc.VectorSubcoreMesh(
    core_axis_name="c", subcore_axis_name="s", num_cores=NC, num_subcores=NS)


def _mask_cols(row_v, col_v, colm_v):
  for j in range(BATCH // LANES):
    sl = pl.ds(j * LANES, LANES)
    r = row_v[sl]
    c = col_v[sl]
    dummy = jnp.int32(N) + lax.rem(r, jnp.int32(NDUM))
    colm_v[sl] = jnp.where(r == c, dummy, c)


def _split(cid, sid, s0, e_pad, nslot):
  epw0 = s0 // NS
  epw1 = (e_pad - s0) // NS
  start = jnp.where(cid == 0, sid * epw0, s0 + sid * epw1)
  niter = jnp.where(cid == 0, epw0 // (BATCH * nslot), epw1 // (BATCH * nslot))
  return start, niter


def _deg_body(s0, e_pad, ei, zeros128, ones128, out, *scr):
  row_v = scr[0:NSLOT]
  col_v = scr[NSLOT:2 * NSLOT]
  colm_v = scr[2 * NSLOT:3 * NSLOT]
  ones_v = scr[3 * NSLOT]
  acc = scr[3 * NSLOT + 1]
  sem_i = scr[3 * NSLOT + 2:3 * NSLOT + 2 + NSLOT]
  sem_s = scr[3 * NSLOT + 2 + NSLOT:3 * NSLOT + 2 + 2 * NSLOT]

  cid = lax.axis_index("c")
  sid = lax.axis_index("s")
  my_rows = pl.ds(sid * RPT, RPT)
  pltpu.sync_copy(zeros128, acc.at[my_rows])
  pltpu.sync_copy(ones128, ones_v)
  plsc.subcore_barrier()

  start, niter = _split(cid, sid, s0, e_pad, NSLOT)

  def ebase(b):
    return pl.multiple_of(start + b * BATCH, BATCH)

  def fire_idx(b, j):
    base = ebase(b)
    pltpu.async_copy(ei.at[0, pl.ds(base, BATCH)], row_v[j], sem_i[j])
    pltpu.async_copy(ei.at[1, pl.ds(base, BATCH)], col_v[j], sem_i[j])

  def wait_idx(b, j):
    base = ebase(b)
    pltpu.make_async_copy(ei.at[0, pl.ds(base, BATCH)], row_v[j], sem_i[j]).wait()
    pltpu.make_async_copy(ei.at[1, pl.ds(base, BATCH)], col_v[j], sem_i[j]).wait()

  for j in range(NSLOT):
    fire_idx(j, j)

  def body(i, carry):
    b0 = i * NSLOT
    scats = []
    for j in range(NSLOT):
      wait_idx(b0 + j, j)
      _mask_cols(row_v[j], col_v[j], colm_v[j])
      scats.append(
          pltpu.async_copy(ones_v, acc.at[colm_v[j]], sem_s[j], add=True))

      @pl.when(i < niter - 1)
      def _(j=j):
        fire_idx(b0 + NSLOT + j, j)

    for dsc in scats:
      dsc.wait()
    return carry

  lax.fori_loop(0, niter, body, 0)
  plsc.subcore_barrier()
  pltpu.sync_copy(acc.at[my_rows], out.at[cid, my_rows])


def _agg_body(s0, e_pad, hp, ei, zeros128, out, *scr):
  ns = NSLOT_A
  row_v = scr[0:ns]
  col_v = scr[ns:2 * ns]
  colm_v = scr[2 * ns:3 * ns]
  rows_v = scr[3 * ns:4 * ns]
  acc = scr[4 * ns]
  sem_i = scr[4 * ns + 1:4 * ns + 1 + ns]
  sem_g = scr[4 * ns + 1 + ns:4 * ns + 1 + 2 * ns]
  sem_s = scr[4 * ns + 1 + 2 * ns:4 * ns + 1 + 3 * ns]

  cid = lax.axis_index("c")
  sid = lax.axis_index("s")
  my_rows = pl.ds(sid * RPT, RPT)
  pltpu.sync_copy(zeros128, acc.at[my_rows])
  plsc.subcore_barrier()

  start, niter = _split(cid, sid, s0, e_pad, ns)

  def ebase(b):
    return pl.multiple_of(start + b * BATCH, BATCH)

  def fire_idx(b, j):
    base = ebase(b)
    pltpu.async_copy(ei.at[0, pl.ds(base, BATCH)], row_v[j], sem_i[j])
    pltpu.async_copy(ei.at[1, pl.ds(base, BATCH)], col_v[j], sem_i[j])

  def wait_idx(b, j):
    base = ebase(b)
    pltpu.make_async_copy(ei.at[0, pl.ds(base, BATCH)], row_v[j], sem_i[j]).wait()
    pltpu.make_async_copy(ei.at[1, pl.ds(base, BATCH)], col_v[j], sem_i[j]).wait()

  for j in range(ns):
    fire_idx(j, j)

  def body(i, carry):
    b0 = i * ns
    gats = []
    for j in range(ns):
      wait_idx(b0 + j, j)
      _mask_cols(row_v[j], col_v[j], colm_v[j])
      gats.append(pltpu.async_copy(hp.at[row_v[j]], rows_v[j], sem_g[j]))

    scats = []
    for j in range(ns):
      gats[j].wait()
      scats.append(
          pltpu.async_copy(rows_v[j], acc.at[colm_v[j]], sem_s[j], add=True))

      @pl.when(i < niter - 1)
      def _(j=j):
        fire_idx(b0 + ns + j, j)

    for dsc in scats:
      dsc.wait()
    return carry

  lax.fori_loop(0, niter, body, 0)
  plsc.subcore_barrier()
  pltpu.sync_copy(acc.at[my_rows], out.at[cid, my_rows])


def _dis_body(degp_ref, out_ref):
  p = degp_ref[...]
  deg = jnp.sum(p[0], axis=-1) + jnp.sum(p[1], axis=-1) + 1.0
  out_ref[...] = lax.rsqrt(deg)[:, None]


def _mm_scale_body(x_ref, w_ref, dis_ref, out_ref):
  h = jnp.dot(x_ref[...], w_ref[...], preferred_element_type=jnp.float32)
  out_ref[...] = h * dis_ref[...]


def _norm_body(aggp_ref, hp_ref, dis_ref, b_ref, g_ref, be_ref, out_ref):
  a = aggp_ref[0] + aggp_ref[1] + hp_ref[...]
  t = a * dis_ref[...] + b_ref[...]
  r = jnp.maximum(t, 0.0)
  mu = jnp.mean(r, axis=-1, keepdims=True)
  c = r - mu
  var = jnp.mean(c * c, axis=-1, keepdims=True)
  out_ref[...] = c * lax.rsqrt(var + EPS) * g_ref[...] + be_ref[...]


def _make_sc_kernels(e_pad, s0_agg=None):
  if s0_agg is None:
    s0_agg = e_pad // 2
  deg_k = functools.partial(
      pl.kernel,
      out_type=jax.ShapeDtypeStruct((NC, N_ACC, D), jnp.float32),
      mesh=_mesh,
      scratch_types=(
          [pltpu.VMEM((BATCH,), jnp.int32)] * (3 * NSLOT)
          + [pltpu.VMEM((BATCH, D), jnp.float32),
             pltpu.VMEM_SHARED((N_ACC, D), jnp.float32)]
          + [pltpu.SemaphoreType.DMA] * (2 * NSLOT)
      ))(functools.partial(_deg_body, e_pad // 2, e_pad))
  agg_k = functools.partial(
      pl.kernel,
      out_type=jax.ShapeDtypeStruct((NC, N_ACC, D), jnp.float32),
      mesh=_mesh,
      scratch_types=(
          [pltpu.VMEM((BATCH,), jnp.int32)] * (3 * NSLOT_A)
          + [pltpu.VMEM((BATCH, D), jnp.float32)] * NSLOT_A
          + [pltpu.VMEM_SHARED((N_ACC, D), jnp.float32)]
          + [pltpu.SemaphoreType.DMA] * (3 * NSLOT_A)
      ))(functools.partial(_agg_body, s0_agg, e_pad))
  return deg_k, agg_k


def _tc_dis(degp):
  return pl.pallas_call(
      _dis_body,
      grid=(N // BLK,),
      in_specs=[pl.BlockSpec((NC, BLK, D), lambda i: (0, i, 0))],
      out_specs=pl.BlockSpec((BLK, 1), lambda i: (i, 0)),
      out_shape=jax.ShapeDtypeStruct((N, 1), jnp.float32),
  )(degp)


def _tc_matmul_scale(x, w, dis):
  return pl.pallas_call(
      _mm_scale_body,
      grid=(N // BLK,),
      in_specs=[
          pl.BlockSpec((BLK, D), lambda i: (i, 0)),
          pl.BlockSpec((D, D), lambda i: (0, 0)),
          pl.BlockSpec((BLK, 1), lambda i: (i, 0)),
      ],
      out_specs=pl.BlockSpec((BLK, D), lambda i: (i, 0)),
      out_shape=jax.ShapeDtypeStruct((N, D), jnp.float32),
  )(x, w, dis)


def _tc_norm(aggp, hp, dis, b, g, be):
  return pl.pallas_call(
      _norm_body,
      grid=(N // BLK,),
      in_specs=[
          pl.BlockSpec((NC, BLK, D), lambda i: (0, i, 0)),
          pl.BlockSpec((BLK, D), lambda i: (i, 0)),
          pl.BlockSpec((BLK, 1), lambda i: (i, 0)),
          pl.BlockSpec((1, D), lambda i: (0, 0)),
          pl.BlockSpec((1, D), lambda i: (0, 0)),
          pl.BlockSpec((1, D), lambda i: (0, 0)),
      ],
      out_specs=pl.BlockSpec((BLK, D), lambda i: (i, 0)),
      out_shape=jax.ShapeDtypeStruct((N, D), jnp.float32),
  )(aggp, hp, dis, b, g, be)


def kernel(x, edge_index, W1, b1, g1, be1, W2, b2, g2, be2):
  e = edge_index.shape[1]
  chunk = NW * BATCH * NSLOT
  e_pad = ((e + chunk - 1) // chunk) * chunk
  pad = e_pad - e
  ei = edge_index.astype(jnp.int32)
  if pad:
    fill = jnp.stack([
        jnp.zeros((pad,), jnp.int32),
        N + (jnp.arange(pad, dtype=jnp.int32) % NDUM)])
    ei = jnp.concatenate([ei, fill], axis=1)

  ones128 = jnp.full((BATCH, D), 1.0 / D, jnp.float32)
  zeros128 = jnp.zeros((RPT, D), jnp.float32)
  b1r, g1r, be1r = b1.reshape(1, D), g1.reshape(1, D), be1.reshape(1, D)
  b2r, g2r, be2r = b2.reshape(1, D), g2.reshape(1, D), be2.reshape(1, D)

  s0_agg = int(round(0.9 * e_pad / 8192)) * 8192
  deg_k, agg_k = _make_sc_kernels(e_pad, s0_agg)

  degp = deg_k(ei, zeros128, ones128)
  dis = _tc_dis(degp)

  h1 = _tc_matmul_scale(x, W1, dis)
  agg1 = agg_k(h1, ei, zeros128)
  z1 = _tc_norm(agg1, h1, dis, b1r, g1r, be1r)

  h2 = _tc_matmul_scale(z1, W2, dis)
  agg2 = agg_k(h2, ei, zeros128)
  z2 = _tc_norm(agg2, h2, dis, b2r, g2r, be2r)
  return z2
